# baseline (speedup 1.0000x reference)
import functools

import jax
import jax.numpy as jnp
from jax import lax
from jax.experimental import pallas as pl
from jax.experimental.pallas import tpu as pltpu
from jax.experimental.pallas import tpu_sc as plsc

_RB = 1024
_NC = 2
_NS = 16
_NW = _NC * _NS
_CHUNK = 2048 // _NW


def _cumsum_lanes(m):
    r, n = m.shape
    c = m
    k = 1
    while k < n:
        c = c + jnp.concatenate(
            [jnp.zeros((r, k), c.dtype), c[:, :n - k]], axis=1)
        k *= 2
    return c


def _route_kernel(x_ref, wg_ref, ppos1_ref, ppos2_ref, w1s_ref, w2s_ref,
                  bexp_ref, nact_ref):
    E = wg_ref.shape[1]
    N = x_ref.shape[0]
    logitsT = jax.lax.dot_general(
        wg_ref[...], x_ref[...],
        dimension_numbers=(((0,), (1,)), ((), ())),
        preferred_element_type=jnp.float32)
    ecol = jax.lax.broadcasted_iota(jnp.int32, (E, N), 0)
    m1 = jnp.max(logitsT, axis=0, keepdims=True)
    e1 = jnp.min(jnp.where(logitsT == m1, ecol, E), axis=0, keepdims=True)
    is1 = ecol == e1
    masked = jnp.where(is1, -jnp.inf, logitsT)
    m2 = jnp.max(masked, axis=0, keepdims=True)
    e2 = jnp.min(jnp.where(masked == m2, ecol, E), axis=0, keepdims=True)
    is2 = ecol == e2
    z = jnp.exp(m2 - m1)
    w1s_ref[...] = 1.0 / (1.0 + z)
    w2s_ref[...] = z / (1.0 + z)

    m1f = is1.astype(jnp.float32)
    m2f = is2.astype(jnp.float32)
    c1 = _cumsum_lanes(m1f)
    c2 = _cumsum_lanes(m2f)
    tot1 = c1[:, N - 1:N]
    tot2 = c2[:, N - 1:N]
    count = tot1 + tot2
    pblocks = jnp.ceil(count / _RB)
    lt = (jax.lax.broadcasted_iota(jnp.int32, (E, E), 0)
          > jax.lax.broadcasted_iota(jnp.int32, (E, E), 1)).astype(jnp.float32)
    pstart = jax.lax.dot_general(
        lt, pblocks, dimension_numbers=(((1,), (0,)), ((), ())),
        preferred_element_type=jnp.float32)
    pad_off = pstart * float(_RB)

    r1 = jnp.sum(m1f * (c1 - 1.0 + pad_off), axis=0, keepdims=True)
    r2 = jnp.sum(m2f * (c2 - 1.0 + tot1 + pad_off), axis=0, keepdims=True)
    ppos1_ref[...] = r1.astype(jnp.int32)
    ppos2_ref[...] = r2.astype(jnp.int32)

    nb = bexp_ref.shape[1]
    bi = jax.lax.broadcasted_iota(jnp.int32, (E, nb), 1).astype(jnp.float32)
    ge = (bi >= pstart).astype(jnp.float32)
    bexp_ref[...] = jnp.clip(
        (jnp.sum(ge, axis=0, keepdims=True) - 1.0).astype(jnp.int32),
        0, E - 1)
    nact_ref[...] = jnp.broadcast_to(
        jnp.sum(pblocks, axis=0, keepdims=True),
        nact_ref.shape).astype(jnp.int32)


def _dispatch_body(x_hbm, p1_hbm, p2_hbm, xs_hbm, idx1_v, idx2_v, xbuf_v, sem):
    wid = lax.axis_index("s") * _NC + lax.axis_index("c")
    base = wid * _CHUNK
    pltpu.sync_copy(p1_hbm.at[0, pl.ds(base, _CHUNK)], idx1_v)
    pltpu.sync_copy(p2_hbm.at[0, pl.ds(base, _CHUNK)], idx2_v)
    pltpu.sync_copy(x_hbm.at[pl.ds(base, _CHUNK)], xbuf_v)
    c1 = pltpu.async_copy(xbuf_v, xs_hbm.at[idx1_v], sem)
    c2 = pltpu.async_copy(xbuf_v, xs_hbm.at[idx2_v], sem)
    c1.wait()
    c2.wait()


def _combine_body(ys_hbm, p1_hbm, p2_hbm, out_hbm,
                  idx1_v, idx2_v, bufa_v, bufb_v, sem):
    wid = lax.axis_index("s") * _NC + lax.axis_index("c")
    half = _CHUNK // 2
    for sub in range(2):
        b2 = wid * _CHUNK + sub * half
        pltpu.sync_copy(p1_hbm.at[0, pl.ds(b2, half)], idx1_v)
        pltpu.sync_copy(p2_hbm.at[0, pl.ds(b2, half)], idx2_v)
        pltpu.async_copy(ys_hbm.at[idx1_v], bufa_v, sem).wait()
        pltpu.async_copy(ys_hbm.at[idx2_v], bufb_v, sem).wait()
        nlane = bufa_v.shape[1] // 16

        def _row_add(r, carry):
            for k in range(nlane):
                sl = pl.ds(k * 16, 16)
                bufa_v[r, sl] = bufa_v[r, sl] + bufb_v[r, sl]
            return carry

        lax.fori_loop(0, half, _row_add, 0)
        pltpu.sync_copy(bufa_v, out_hbm.at[pl.ds(b2, half)])


def _gmm_kernel(bexp_ref, nact_ref, xs_ref, w1_ref, w2_ref,
                p1_ref, p2_ref, w1s_ref, w2s_ref, ys_ref):
    i = pl.program_id(0)

    @pl.when(i < nact_ref[0, 0])
    def _():
        h = jnp.maximum(
            jnp.dot(xs_ref[...], w1_ref[0],
                    preferred_element_type=jnp.float32), 0.0)
        y = jnp.dot(h, w2_ref[0], preferred_element_type=jnp.float32)
        RB, n = xs_ref.shape[0], p1_ref.shape[1]
        rows = jax.lax.broadcasted_iota(jnp.int32, (RB, n), 0) + i * RB
        wrow = (jnp.sum(jnp.where(rows == p1_ref[...], w1s_ref[...], 0.0),
                        axis=1, keepdims=True)
                + jnp.sum(jnp.where(rows == p2_ref[...], w2s_ref[...], 0.0),
                          axis=1, keepdims=True))
        ys_ref[...] = y * wrow


def kernel(x, Wg, W1, W2):
    B, T, C = x.shape
    N = B * T
    E = Wg.shape[1]
    DFF = W1.shape[2]
    NBMAX = (N * 2) // _RB + E
    NKPAD = NBMAX * _RB
    xf = x.reshape(N, C)

    ppos1, ppos2, w1s, w2s, bexp, nact = pl.pallas_call(
        _route_kernel,
        in_specs=[
            pl.BlockSpec((N, C), lambda: (0, 0)),
            pl.BlockSpec((C, E), lambda: (0, 0)),
        ],
        out_specs=[
            pl.BlockSpec((1, N), lambda: (0, 0)),
            pl.BlockSpec((1, N), lambda: (0, 0)),
            pl.BlockSpec((1, N), lambda: (0, 0)),
            pl.BlockSpec((1, N), lambda: (0, 0)),
            pl.BlockSpec((1, 128), lambda: (0, 0)),
            pl.BlockSpec((1, 8), lambda: (0, 0)),
        ],
        out_shape=[
            jax.ShapeDtypeStruct((1, N), jnp.int32),
            jax.ShapeDtypeStruct((1, N), jnp.int32),
            jax.ShapeDtypeStruct((1, N), jnp.float32),
            jax.ShapeDtypeStruct((1, N), jnp.float32),
            jax.ShapeDtypeStruct((1, 128), jnp.int32),
            jax.ShapeDtypeStruct((1, 8), jnp.int32),
        ],
    )(xf, Wg)

    mesh = plsc.VectorSubcoreMesh(
        core_axis_name="c", subcore_axis_name="s",
        num_cores=_NC, num_subcores=_NS)

    xs = pl.kernel(
        _dispatch_body,
        out_type=jax.ShapeDtypeStruct((NKPAD, C), jnp.float32),
        mesh=mesh,
        scratch_types=[
            pltpu.VMEM((_CHUNK,), jnp.int32),
            pltpu.VMEM((_CHUNK,), jnp.int32),
            pltpu.VMEM((_CHUNK, C), jnp.float32),
            pltpu.SemaphoreType.DMA,
        ],
    )(xf, ppos1, ppos2)

    ys = pl.pallas_call(
        _gmm_kernel,
        grid_spec=pltpu.PrefetchScalarGridSpec(
            num_scalar_prefetch=2,
            grid=(NBMAX,),
            in_specs=[
                pl.BlockSpec((_RB, C), lambda i, be, na: (i, 0)),
                pl.BlockSpec((1, C, DFF), lambda i, be, na: (be[0, i], 0, 0)),
                pl.BlockSpec((1, DFF, C), lambda i, be, na: (be[0, i], 0, 0)),
                pl.BlockSpec((1, N), lambda i, be, na: (0, 0)),
                pl.BlockSpec((1, N), lambda i, be, na: (0, 0)),
                pl.BlockSpec((1, N), lambda i, be, na: (0, 0)),
                pl.BlockSpec((1, N), lambda i, be, na: (0, 0)),
            ],
            out_specs=pl.BlockSpec((_RB, C), lambda i, be, na: (i, 0)),
        ),
        out_shape=jax.ShapeDtypeStruct((NKPAD, C), jnp.float32),
    )(bexp, nact, xs, W1, W2, ppos1, ppos2, w1s, w2s)

    out = pl.kernel(
        _combine_body,
        out_type=jax.ShapeDtypeStruct((N, C), jnp.float32),
        mesh=mesh,
        scratch_types=[
            pltpu.VMEM((_CHUNK // 2,), jnp.int32),
            pltpu.VMEM((_CHUNK // 2,), jnp.int32),
            pltpu.VMEM((_CHUNK // 2, C), jnp.float32),
            pltpu.VMEM((_CHUNK // 2, C), jnp.float32),
            pltpu.SemaphoreType.DMA,
        ],
    )(ys, ppos1, ppos2)
    return out.reshape(B, T, C)


# speedup vs baseline: 1.1115x; 1.1115x over previous
import functools

import jax
import jax.numpy as jnp
from jax import lax
from jax.experimental import pallas as pl
from jax.experimental.pallas import tpu as pltpu
from jax.experimental.pallas import tpu_sc as plsc

_RB = 512
_NC = 2
_NS = 16
_NW = _NC * _NS
_CHUNK = 2048 // _NW


def _cumsum_lanes(m):
    r, n = m.shape
    c = m
    k = 1
    while k < n:
        c = c + jnp.concatenate(
            [jnp.zeros((r, k), c.dtype), c[:, :n - k]], axis=1)
        k *= 2
    return c


def _route_kernel(x_ref, wg_ref, ppos1_ref, ppos2_ref, w1s_ref, w2s_ref,
                  bexp_ref, nact_ref):
    E = wg_ref.shape[1]
    N = x_ref.shape[0]
    logitsT = jax.lax.dot_general(
        wg_ref[...], x_ref[...],
        dimension_numbers=(((0,), (1,)), ((), ())),
        preferred_element_type=jnp.float32)
    ecol = jax.lax.broadcasted_iota(jnp.int32, (E, N), 0)
    m1 = jnp.max(logitsT, axis=0, keepdims=True)
    e1 = jnp.min(jnp.where(logitsT == m1, ecol, E), axis=0, keepdims=True)
    is1 = ecol == e1
    masked = jnp.where(is1, -jnp.inf, logitsT)
    m2 = jnp.max(masked, axis=0, keepdims=True)
    e2 = jnp.min(jnp.where(masked == m2, ecol, E), axis=0, keepdims=True)
    is2 = ecol == e2
    z = jnp.exp(m2 - m1)
    w1s_ref[...] = 1.0 / (1.0 + z)
    w2s_ref[...] = z / (1.0 + z)

    m1f = is1.astype(jnp.float32)
    m2f = is2.astype(jnp.float32)
    c1 = _cumsum_lanes(m1f)
    c2 = _cumsum_lanes(m2f)
    tot1 = c1[:, N - 1:N]
    tot2 = c2[:, N - 1:N]
    count = tot1 + tot2
    pblocks = jnp.ceil(count / _RB)
    lt = (jax.lax.broadcasted_iota(jnp.int32, (E, E), 0)
          > jax.lax.broadcasted_iota(jnp.int32, (E, E), 1)).astype(jnp.float32)
    pstart = jax.lax.dot_general(
        lt, pblocks, dimension_numbers=(((1,), (0,)), ((), ())),
        preferred_element_type=jnp.float32)
    pad_off = pstart * float(_RB)

    r1 = jnp.sum(m1f * (c1 - 1.0 + pad_off), axis=0, keepdims=True)
    r2 = jnp.sum(m2f * (c2 - 1.0 + tot1 + pad_off), axis=0, keepdims=True)
    ppos1_ref[...] = r1.astype(jnp.int32)
    ppos2_ref[...] = r2.astype(jnp.int32)

    nb = bexp_ref.shape[1]
    bi = jax.lax.broadcasted_iota(jnp.int32, (E, nb), 1).astype(jnp.float32)
    ge = (bi >= pstart).astype(jnp.float32)
    bexp_ref[...] = jnp.clip(
        (jnp.sum(ge, axis=0, keepdims=True) - 1.0).astype(jnp.int32),
        0, E - 1)
    nact_ref[...] = jnp.broadcast_to(
        jnp.sum(pblocks, axis=0, keepdims=True),
        nact_ref.shape).astype(jnp.int32)


def _dispatch_body(x_hbm, p1_hbm, p2_hbm, xs_hbm, idx1_v, idx2_v, xbuf_v, sem):
    wid = lax.axis_index("s") * _NC + lax.axis_index("c")
    base = wid * _CHUNK
    pltpu.sync_copy(p1_hbm.at[0, pl.ds(base, _CHUNK)], idx1_v)
    pltpu.sync_copy(p2_hbm.at[0, pl.ds(base, _CHUNK)], idx2_v)
    pltpu.sync_copy(x_hbm.at[pl.ds(base, _CHUNK)], xbuf_v)
    c1 = pltpu.async_copy(xbuf_v, xs_hbm.at[idx1_v], sem)
    c2 = pltpu.async_copy(xbuf_v, xs_hbm.at[idx2_v], sem)
    c1.wait()
    c2.wait()


def _combine_body(ys_hbm, p1_hbm, p2_hbm, out_hbm,
                  idx1_v, idx2_v, bufa_v, bufb_v, sem):
    wid = lax.axis_index("s") * _NC + lax.axis_index("c")
    half = _CHUNK // 2
    for sub in range(2):
        b2 = wid * _CHUNK + sub * half
        pltpu.sync_copy(p1_hbm.at[0, pl.ds(b2, half)], idx1_v)
        pltpu.sync_copy(p2_hbm.at[0, pl.ds(b2, half)], idx2_v)
        pltpu.async_copy(ys_hbm.at[idx1_v], bufa_v, sem).wait()
        pltpu.async_copy(ys_hbm.at[idx2_v], bufb_v, sem).wait()
        nlane = bufa_v.shape[1] // 16

        def _row_add(r, carry):
            for k in range(nlane):
                sl = pl.ds(k * 16, 16)
                bufa_v[r, sl] = bufa_v[r, sl] + bufb_v[r, sl]
            return carry

        lax.fori_loop(0, half, _row_add, 0)
        pltpu.sync_copy(bufa_v, out_hbm.at[pl.ds(b2, half)])


def _gmm_kernel(bexp_ref, nact_ref, xs_ref, w1_ref, w2_ref,
                p1_ref, p2_ref, w1s_ref, w2s_ref, ys_ref):
    i = pl.program_id(0)

    @pl.when(i < nact_ref[0, 0])
    def _():
        h = jnp.maximum(
            jnp.dot(xs_ref[...], w1_ref[0],
                    preferred_element_type=jnp.float32), 0.0)
        y = jnp.dot(h, w2_ref[0], preferred_element_type=jnp.float32)
        RB, n = xs_ref.shape[0], p1_ref.shape[1]
        rows = jax.lax.broadcasted_iota(jnp.int32, (RB, n), 0) + i * RB
        wrow = (jnp.sum(jnp.where(rows == p1_ref[...], w1s_ref[...], 0.0),
                        axis=1, keepdims=True)
                + jnp.sum(jnp.where(rows == p2_ref[...], w2s_ref[...], 0.0),
                          axis=1, keepdims=True))
        ys_ref[...] = y * wrow


def kernel(x, Wg, W1, W2):
    B, T, C = x.shape
    N = B * T
    E = Wg.shape[1]
    DFF = W1.shape[2]
    NBMAX = (N * 2) // _RB + E
    NKPAD = NBMAX * _RB
    xf = x.reshape(N, C)

    ppos1, ppos2, w1s, w2s, bexp, nact = pl.pallas_call(
        _route_kernel,
        in_specs=[
            pl.BlockSpec((N, C), lambda: (0, 0)),
            pl.BlockSpec((C, E), lambda: (0, 0)),
        ],
        out_specs=[
            pl.BlockSpec((1, N), lambda: (0, 0)),
            pl.BlockSpec((1, N), lambda: (0, 0)),
            pl.BlockSpec((1, N), lambda: (0, 0)),
            pl.BlockSpec((1, N), lambda: (0, 0)),
            pl.BlockSpec((1, 128), lambda: (0, 0)),
            pl.BlockSpec((1, 8), lambda: (0, 0)),
        ],
        out_shape=[
            jax.ShapeDtypeStruct((1, N), jnp.int32),
            jax.ShapeDtypeStruct((1, N), jnp.int32),
            jax.ShapeDtypeStruct((1, N), jnp.float32),
            jax.ShapeDtypeStruct((1, N), jnp.float32),
            jax.ShapeDtypeStruct((1, 128), jnp.int32),
            jax.ShapeDtypeStruct((1, 8), jnp.int32),
        ],
    )(xf, Wg)

    mesh = plsc.VectorSubcoreMesh(
        core_axis_name="c", subcore_axis_name="s",
        num_cores=_NC, num_subcores=_NS)

    xs = pl.kernel(
        _dispatch_body,
        out_type=jax.ShapeDtypeStruct((NKPAD, C), jnp.float32),
        mesh=mesh,
        scratch_types=[
            pltpu.VMEM((_CHUNK,), jnp.int32),
            pltpu.VMEM((_CHUNK,), jnp.int32),
            pltpu.VMEM((_CHUNK, C), jnp.float32),
            pltpu.SemaphoreType.DMA,
        ],
    )(xf, ppos1, ppos2)

    ys = pl.pallas_call(
        _gmm_kernel,
        grid_spec=pltpu.PrefetchScalarGridSpec(
            num_scalar_prefetch=2,
            grid=(NBMAX,),
            in_specs=[
                pl.BlockSpec((_RB, C), lambda i, be, na: (i, 0)),
                pl.BlockSpec((1, C, DFF), lambda i, be, na: (be[0, i], 0, 0)),
                pl.BlockSpec((1, DFF, C), lambda i, be, na: (be[0, i], 0, 0)),
                pl.BlockSpec((1, N), lambda i, be, na: (0, 0)),
                pl.BlockSpec((1, N), lambda i, be, na: (0, 0)),
                pl.BlockSpec((1, N), lambda i, be, na: (0, 0)),
                pl.BlockSpec((1, N), lambda i, be, na: (0, 0)),
            ],
            out_specs=pl.BlockSpec((_RB, C), lambda i, be, na: (i, 0)),
        ),
        out_shape=jax.ShapeDtypeStruct((NKPAD, C), jnp.float32),
    )(bexp, nact, xs, W1, W2, ppos1, ppos2, w1s, w2s)

    out = pl.kernel(
        _combine_body,
        out_type=jax.ShapeDtypeStruct((N, C), jnp.float32),
        mesh=mesh,
        scratch_types=[
            pltpu.VMEM((_CHUNK // 2,), jnp.int32),
            pltpu.VMEM((_CHUNK // 2,), jnp.int32),
            pltpu.VMEM((_CHUNK // 2, C), jnp.float32),
            pltpu.VMEM((_CHUNK // 2, C), jnp.float32),
            pltpu.SemaphoreType.DMA,
        ],
    )(ys, ppos1, ppos2)
    return out.reshape(B, T, C)
